# baseline (speedup 1.0000x reference)
import functools

import jax
import jax.numpy as jnp
from jax.experimental import pallas as pl
from jax.experimental.pallas import tpu as pltpu


def _dense_moe_body(x_ref, wr_ref, br_ref, w1_ref, b1_ref, w2_ref, b2_ref,
                    out_ref, acc_ref, *, n_experts, block_t):
    e = pl.program_id(0)
    t = pl.program_id(1)
    x = x_ref[...]

    logits = jnp.dot(x, wr_ref[...], preferred_element_type=jnp.float32)
    logits = logits + br_ref[...]
    lm = jnp.max(logits, axis=-1, keepdims=True)
    ex = jnp.exp(logits - lm)
    p = ex / jnp.sum(ex, axis=-1, keepdims=True)
    iota = jax.lax.broadcasted_iota(jnp.int32, p.shape, 1)
    m1 = jnp.max(p, axis=-1, keepdims=True)
    i1 = jnp.min(jnp.where(p == m1, iota, n_experts), axis=-1, keepdims=True)
    pneg = jnp.where(iota == i1, -jnp.inf, p)
    m2 = jnp.max(pneg, axis=-1, keepdims=True)
    i2 = jnp.min(jnp.where(pneg == m2, iota, n_experts), axis=-1, keepdims=True)
    gate = (jnp.where(i1 == e, m1, 0.0)
            + jnp.where(i2 == e, m2, 0.0))

    h = jnp.dot(x, w1_ref[0], preferred_element_type=jnp.float32)
    h = jnp.maximum(h + b1_ref[0], 0.0)
    y = jnp.dot(h, w2_ref[0], preferred_element_type=jnp.float32)
    y = y + b2_ref[0]
    contrib = gate * y

    rows = pl.ds(t * block_t, block_t)
    prev = jnp.where(e == 0, 0.0, acc_ref[rows, :])
    acc = prev + contrib
    acc_ref[rows, :] = acc
    out_ref[...] = acc


def kernel(x, Wr, br, W1, b1, W2, b2):
    T, D = x.shape
    E = Wr.shape[1]
    F = W1.shape[2]
    BT = 256
    NT = T // BT

    body = functools.partial(_dense_moe_body, n_experts=E, block_t=BT)
    out = pl.pallas_call(
        body,
        grid=(E, NT),
        in_specs=[
            pl.BlockSpec((BT, D), lambda e, t: (t, 0)),
            pl.BlockSpec((D, E), lambda e, t: (0, 0)),
            pl.BlockSpec((1, E), lambda e, t: (0, 0)),
            pl.BlockSpec((1, D, F), lambda e, t: (e, 0, 0)),
            pl.BlockSpec((1, 1, F), lambda e, t: (e, 0, 0)),
            pl.BlockSpec((1, F, D), lambda e, t: (e, 0, 0)),
            pl.BlockSpec((1, 1, D), lambda e, t: (e, 0, 0)),
        ],
        out_specs=pl.BlockSpec((BT, D), lambda e, t: (t, 0)),
        out_shape=jax.ShapeDtypeStruct((T, D), jnp.float32),
        scratch_shapes=[pltpu.VMEM((T, D), jnp.float32)],
    )(x, Wr, br.reshape(1, E), W1, b1.reshape(E, 1, F), W2, b2.reshape(E, 1, D))
    return out
